# baseline (speedup 1.0000x reference)
import dataclasses
import functools

import jax
import jax.numpy as jnp
from jax import lax
from jax.experimental import pallas as pl
from jax.experimental.pallas import tpu as pltpu
from jax.experimental.pallas import tpu_sc as plsc

_LANES = 16
_HIST = 50
_BROWS = 4
_BLK = _BROWS * _HIST
_BLK_PAD = 256
_WORK = 208
_NCHUNK = _WORK // _LANES
_GATHERS = ((0, 128), (128, 72))
_RING = 3


def _embedding_gather(wide_table, ids_pad, batch, dim):
    nblocks = ids_pad.shape[0]
    mesh = plsc.VectorSubcoreMesh(core_axis_name="core", subcore_axis_name="subcore")
    cp = pltpu.CompilerParams()
    if "needs_layout_passes" in pltpu.CompilerParams.__dataclass_fields__:
        cp = dataclasses.replace(cp, needs_layout_passes=False)

    @pl.kernel(
        compiler_params=cp,
        out_type=jax.ShapeDtypeStruct((batch, _HIST, dim), jnp.float32),
        mesh=mesh,
        scratch_types=(
            [pltpu.VMEM((_WORK,), jnp.int32) for _ in range(_RING)]
            + [pltpu.VMEM((_BLK, 4 * dim), jnp.float32) for _ in range(_RING)]
            + [
                pltpu.VMEM((_WORK,), jnp.int32),
                pltpu.VMEM((_WORK,), jnp.int32),
                pltpu.SemaphoreType.DMA((_RING,)),
                pltpu.SMEM((1,), jnp.int32),
            ]
        ),
    )
    def kernel_fn(table_hbm, idx_hbm, out_hbm, *scratch):
        gidx = scratch[:_RING]
        bufs = scratch[_RING : 2 * _RING]
        rb_v, rt_v, g_sem, cnt = scratch[2 * _RING :]
        iota = jax.lax.iota(jnp.int32, _LANES)
        cnt[0] = 0
        steps = nblocks // 32

        @pl.loop(0, _WORK, step=_LANES)
        def _(j):
            rows = j + iota
            b = rows // _HIST
            rb_v[pl.ds(j, _LANES)] = b
            rt_v[pl.ds(j, _LANES)] = rows - b * _HIST

        def idx_math(idx_ref, slot):
            @pl.loop(0, _WORK, step=_LANES)
            def _(c):
                chunk = idx_ref[0, pl.ds(c, _LANES)]
                gidx[slot][pl.ds(c, _LANES)] = jax.lax.shift_right_logical(
                    chunk, 2
                )

        def fire(slot):
            for off, num in _GATHERS:
                pltpu.async_copy(
                    table_hbm.at[gidx[slot].at[pl.ds(off, num)]],
                    bufs[slot].at[pl.ds(off, num)],
                    g_sem.at[slot],
                )

        def wait(slot):
            for off, num in _GATHERS:
                pltpu.make_async_copy(
                    table_hbm.at[gidx[slot].at[pl.ds(off, num)]],
                    bufs[slot].at[pl.ds(off, num)],
                    g_sem.at[slot],
                ).wait()

        def extract(idx_ref, out_vmem, slot):
            for cc in range(_NCHUNK):
                j = cc * _LANES
                valid = _BLK - j
                mask = None if valid >= _LANES else iota < valid
                rows = j + iota
                cb = (idx_ref[0, pl.ds(j, _LANES)] & 3) * dim
                rb = rb_v[pl.ds(j, _LANES)]
                rt = rt_v[pl.ds(j, _LANES)]
                for ci in range(dim):
                    vals = plsc.load_gather(bufs[slot], [rows, cb + ci], mask=mask)
                    plsc.store_scatter(
                        out_vmem,
                        [rb, rt, jnp.full((_LANES,), ci, jnp.int32)],
                        vals,
                        mask=mask,
                    )

        def body(idx_cur, idx_a1, idx_a2, out_vmem):
            g = cnt[0]

            @pl.when(g == 0)
            def _():
                idx_math(idx_cur, 0)
                fire(0)
                idx_math(idx_a1, 1)
                fire(1)

            for slot in range(_RING):

                @pl.when(lax.rem(g, _RING) == slot)
                def _(slot=slot):
                    nxt = (slot + _RING - 1) % _RING

                    @pl.when(g < steps - (_RING - 1))
                    def _():
                        idx_math(idx_a2, nxt)
                        fire(nxt)

                    wait(slot)
                    extract(idx_cur, out_vmem, slot)

            cnt[0] = g + 1

        pltpu.emit_pipeline(
            body,
            grid=(nblocks,),
            in_specs=[
                pl.BlockSpec((1, _BLK_PAD), index_map=lambda i: (i, 0)),
                pl.BlockSpec(
                    (1, _BLK_PAD),
                    index_map=lambda i: (jnp.minimum(i + 1, nblocks - 1), 0),
                ),
                pl.BlockSpec(
                    (1, _BLK_PAD),
                    index_map=lambda i: (jnp.minimum(i + 2, nblocks - 1), 0),
                ),
            ],
            out_specs=[
                pl.BlockSpec((_BROWS, _HIST, dim), index_map=lambda i: (i, 0, 0))
            ],
            core_axis_name=("core", "subcore"),
            dimension_semantics=(pltpu.PARALLEL,),
        )(idx_hbm, idx_hbm, idx_hbm, out_hbm)

    return kernel_fn(wide_table, ids_pad)


def kernel(token_ids, weight):
    batch, hist = token_ids.shape
    num_rows, dim = weight.shape
    wide_table = weight.reshape(num_rows // 4, 4 * dim)
    ids_blk = token_ids.reshape(batch // _BROWS, _BLK)
    ids_pad = jnp.pad(ids_blk, ((0, 0), (0, _BLK_PAD - _BLK)))
    return _embedding_gather(wide_table, ids_pad, batch, dim)


# speedup vs baseline: 1.0041x; 1.0041x over previous
import dataclasses
import functools

import jax
import jax.numpy as jnp
from jax import lax
from jax.experimental import pallas as pl
from jax.experimental.pallas import tpu as pltpu
from jax.experimental.pallas import tpu_sc as plsc

_LANES = 16
_HIST = 50
_BROWS = 4
_BLK = _BROWS * _HIST
_BLK_PAD = 256
_WORK = 208
_NCHUNK = _WORK // _LANES
_GATHERS = ((0, 128), (128, 72))
_RING = 2


def _embedding_gather(wide_table, ids_pad, batch, dim):
    nblocks = ids_pad.shape[0]
    mesh = plsc.VectorSubcoreMesh(core_axis_name="core", subcore_axis_name="subcore")
    cp = pltpu.CompilerParams()
    if "needs_layout_passes" in pltpu.CompilerParams.__dataclass_fields__:
        cp = dataclasses.replace(cp, needs_layout_passes=False)

    @pl.kernel(
        compiler_params=cp,
        out_type=jax.ShapeDtypeStruct((batch, _HIST, dim), jnp.float32),
        mesh=mesh,
        scratch_types=(
            [pltpu.VMEM((_WORK,), jnp.int32) for _ in range(_RING)]
            + [pltpu.VMEM((_BLK, 4 * dim), jnp.float32) for _ in range(_RING)]
            + [
                pltpu.VMEM((_WORK,), jnp.int32),
                pltpu.VMEM((_WORK,), jnp.int32),
                pltpu.SemaphoreType.DMA((_RING,)),
                pltpu.SMEM((1,), jnp.int32),
            ]
        ),
    )
    def kernel_fn(table_hbm, idx_hbm, out_hbm, *scratch):
        gidx = scratch[:_RING]
        bufs = scratch[_RING : 2 * _RING]
        rb_v, rt_v, g_sem, cnt = scratch[2 * _RING :]
        iota = jax.lax.iota(jnp.int32, _LANES)
        cnt[0] = 0
        steps = nblocks // 32

        @pl.loop(0, _WORK, step=_LANES)
        def _(j):
            rows = j + iota
            b = rows // _HIST
            rb_v[pl.ds(j, _LANES)] = b
            rt_v[pl.ds(j, _LANES)] = rows - b * _HIST

        def idx_math(idx_ref, slot):
            @pl.loop(0, _WORK, step=_LANES)
            def _(c):
                chunk = idx_ref[0, pl.ds(c, _LANES)]
                gidx[slot][pl.ds(c, _LANES)] = jax.lax.shift_right_logical(
                    chunk, 2
                )

        def fire(slot):
            for off, num in _GATHERS:
                pltpu.async_copy(
                    table_hbm.at[gidx[slot].at[pl.ds(off, num)]],
                    bufs[slot].at[pl.ds(off, num)],
                    g_sem.at[slot],
                )

        def wait(slot):
            for off, num in _GATHERS:
                pltpu.make_async_copy(
                    table_hbm.at[gidx[slot].at[pl.ds(off, num)]],
                    bufs[slot].at[pl.ds(off, num)],
                    g_sem.at[slot],
                ).wait()

        def extract(idx_ref, out_vmem, slot):
            for cc in range(_NCHUNK):
                j = cc * _LANES
                valid = _BLK - j
                mask = None if valid >= _LANES else iota < valid
                rows = j + iota
                cb = (idx_ref[0, pl.ds(j, _LANES)] & 3) * dim
                rb = rb_v[pl.ds(j, _LANES)]
                rt = rt_v[pl.ds(j, _LANES)]
                for ci in range(dim):
                    vals = plsc.load_gather(bufs[slot], [rows, cb + ci], mask=mask)
                    plsc.store_scatter(
                        out_vmem,
                        [rb, rt, jnp.full((_LANES,), ci, jnp.int32)],
                        vals,
                        mask=mask,
                    )

        def body(idx_cur, idx_a1, out_vmem):
            g = cnt[0]

            @pl.when(g == 0)
            def _():
                idx_math(idx_cur, 0)
                fire(0)

            for slot in range(_RING):

                @pl.when(lax.rem(g, _RING) == slot)
                def _(slot=slot):
                    nxt = (slot + _RING - 1) % _RING

                    @pl.when(g < steps - (_RING - 1))
                    def _():
                        idx_math(idx_a1, nxt)
                        fire(nxt)

                    wait(slot)
                    extract(idx_cur, out_vmem, slot)

            cnt[0] = g + 1

        pltpu.emit_pipeline(
            body,
            grid=(nblocks,),
            in_specs=[
                pl.BlockSpec((1, _BLK_PAD), index_map=lambda i: (i, 0)),
                pl.BlockSpec(
                    (1, _BLK_PAD),
                    index_map=lambda i: (jnp.minimum(i + 1, nblocks - 1), 0),
                ),
            ],
            out_specs=[
                pl.BlockSpec((_BROWS, _HIST, dim), index_map=lambda i: (i, 0, 0))
            ],
            core_axis_name=("core", "subcore"),
            dimension_semantics=(pltpu.PARALLEL,),
        )(idx_hbm, idx_hbm, out_hbm)

    return kernel_fn(wide_table, ids_pad)


def kernel(token_ids, weight):
    batch, hist = token_ids.shape
    num_rows, dim = weight.shape
    wide_table = weight.reshape(num_rows // 4, 4 * dim)
    ids_blk = token_ids.reshape(batch // _BROWS, _BLK)
    ids_pad = jnp.pad(ids_blk, ((0, 0), (0, _BLK_PAD - _BLK)))
    return _embedding_gather(wide_table, ids_pad, batch, dim)
